# Initial kernel scaffold; baseline (speedup 1.0000x reference)
import jax
import jax.numpy as jnp
from jax.experimental import pallas as pl


def kernel(epoch, f_s, f_t, memory_t, idx, contrast_idx):
    raise NotImplementedError("write your pallas kernel here")


# speedup vs baseline: 58.7703x; 58.7703x over previous
import functools

import jax
import jax.numpy as jnp
from jax import lax
from jax.experimental import pallas as pl
from jax.experimental.pallas import tpu as pltpu
from jax.experimental.pallas import tpu_sc as plsc

EPS_ = 1e-07
T_ = 0.07

NC_, NS_ = 2, 16
NW_ = NC_ * NS_
LANES_ = 16
CHUNK_ = 128


def _sc_dots_kernel(B, D, K, bpw, nchunk):
    nd = D // LANES_
    mesh = plsc.VectorSubcoreMesh(
        core_axis_name="c", subcore_axis_name="s",
        num_cores=NC_, num_subcores=NS_)

    @functools.partial(
        pl.kernel,
        out_type=[
            jax.ShapeDtypeStruct((B, K), jnp.float32),
            jax.ShapeDtypeStruct((B,), jnp.float32),
        ],
        mesh=mesh,
        scratch_types=[
            pltpu.VMEM((bpw, nchunk, CHUNK_), jnp.int32),
            pltpu.VMEM((bpw, D), jnp.float32),
            pltpu.VMEM((bpw,), jnp.int32),
            pltpu.VMEM((bpw, D), jnp.float32),
            pltpu.VMEM((2, CHUNK_, D), jnp.float32),
            pltpu.VMEM((bpw, K), jnp.float32),
            pltpu.VMEM((bpw,), jnp.float32),
            pltpu.SemaphoreType.DMA,
            pltpu.SemaphoreType.DMA,
            pltpu.SemaphoreType.DMA,
        ],
    )
    def sc_dots(mem_hbm, fs_hbm, idx_hbm, cidx_hbm, neg_hbm, pos_hbm,
                idx_v, fs_v, pidx_v, prows_v, rows_v, out_v, pout_v,
                sem0, sem1, psem):
        sems = (sem0, sem1)
        wid = lax.axis_index("s") * NC_ + lax.axis_index("c")
        base = wid * bpw
        pltpu.sync_copy(cidx_hbm.at[pl.ds(base, bpw)], idx_v)
        pltpu.sync_copy(fs_hbm.at[pl.ds(base, bpw)], fs_v)
        pltpu.sync_copy(idx_hbm.at[pl.ds(base, bpw)], pidx_v)

        lanes = lax.iota(jnp.int32, LANES_)
        nt = bpw * nchunk

        gdn = lax.GatherDimensionNumbers(
            offset_dims=(), collapsed_slice_dims=(0,), start_index_map=(0,))

        def lanesum(v):
            for s in (8, 4, 2, 1):
                perm = lax.gather(
                    v, (lanes ^ s)[:, None], gdn, (1,),
                    mode=lax.GatherScatterMode.PROMISE_IN_BOUNDS)
                v = v + perm
            return v

        def issue(t, p):
            b = t // nchunk
            j = t % nchunk
            pltpu.async_copy(mem_hbm.at[idx_v.at[b, j]], rows_v.at[p], sems[p])

        issue(jnp.int32(0), 0)
        issue(jnp.int32(1), 1)

        pltpu.async_copy(mem_hbm.at[pidx_v], prows_v, psem).wait()
        for g in range(bpw // LANES_):
            res = jnp.zeros((LANES_,), jnp.float32)
            for r in range(LANES_):
                bl = g * LANES_ + r
                acc = prows_v[bl, pl.ds(0, LANES_)] * fs_v[bl, pl.ds(0, LANES_)]
                for i in range(1, nd):
                    acc = acc + (prows_v[bl, pl.ds(i * LANES_, LANES_)]
                                 * fs_v[bl, pl.ds(i * LANES_, LANES_)])
                res = jnp.where(lanes == r, lanesum(acc), res)
            pout_v[pl.ds(g * LANES_, LANES_)] = res
        pltpu.sync_copy(pout_v, pos_hbm.at[pl.ds(base, bpw)])

        def compute(t, p):
            b = t // nchunk
            j = t % nchunk
            pltpu.make_async_copy(
                mem_hbm.at[idx_v.at[b, j]], rows_v.at[p], sems[p]).wait()
            fv = [fs_v[b, pl.ds(i * LANES_, LANES_)] for i in range(nd)]

            def gbody(g, carry):
                res = jnp.zeros((LANES_,), jnp.float32)
                for r in range(LANES_):
                    row = g * LANES_ + r
                    acc = rows_v[p, row, pl.ds(0, LANES_)] * fv[0]
                    for i in range(1, nd):
                        acc = acc + rows_v[p, row, pl.ds(i * LANES_, LANES_)] * fv[i]
                    res = jnp.where(lanes == r, lanesum(acc), res)
                out_v[b, pl.ds(j * CHUNK_ + g * LANES_, LANES_)] = res
                return carry

            lax.fori_loop(0, CHUNK_ // LANES_, gbody, 0)

        def tbody(m, carry):
            for p in range(2):
                t = 2 * m + p
                compute(t, p)

                @pl.when(t + 2 < nt)
                def _():
                    issue(t + 2, p)
            return carry

        lax.fori_loop(0, nt // 2, tbody, 0)
        pltpu.sync_copy(out_v, neg_hbm.at[pl.ds(base, bpw)])

    return sc_dots


def _loss_kernel(B, K, n_data):
    m = float(K)
    pn = 1.0 / float(n_data)
    c = m * pn

    def body(fs_ref, neg_ref, pos_ref, out_ref):
        fs = fs_ref[...]
        nrm = jnp.sqrt(jnp.sum(fs * fs, axis=1, keepdims=True))
        scale = 1.0 / (nrm * T_)
        e_neg = jnp.exp(neg_ref[...] * scale)
        e_pos = jnp.exp(pos_ref[...] * scale)
        z = ((jnp.sum(e_neg) + jnp.sum(e_pos)) / float(B * (K + 1))) * float(n_data)
        p_pos = e_pos / z
        p_neg = e_neg / z
        ld1 = jnp.sum(jnp.log(p_pos / (p_pos + c + EPS_)))
        ld0 = jnp.sum(jnp.log(c / (p_neg + c + EPS_)))
        out_ref[0, 0] = -(ld1 + ld0) / float(B)

    return pl.pallas_call(
        body,
        out_shape=jax.ShapeDtypeStruct((1, 1), jnp.float32),
        out_specs=pl.BlockSpec(memory_space=pltpu.SMEM),
    )


def kernel(epoch, f_s, f_t, memory_t, idx, contrast_idx):
    del epoch, f_t
    B, D = f_s.shape
    K = contrast_idx.shape[1]
    n_data = memory_t.shape[0]
    bpw = B // NW_
    nchunk = K // CHUNK_

    idx32 = idx.astype(jnp.int32)
    cidx3 = contrast_idx.astype(jnp.int32).reshape(B, nchunk, CHUNK_)

    neg, pos = _sc_dots_kernel(B, D, K, bpw, nchunk)(
        memory_t, f_s, idx32, cidx3)
    loss = _loss_kernel(B, K, n_data)(f_s, neg, pos.reshape(B, 1))
    return loss[0, 0]
